# baseline (speedup 1.0000x reference)
import jax
import jax.numpy as jnp
from jax import lax
from jax.experimental import pallas as pl
from jax.experimental.pallas import tpu as pltpu
from jax.experimental.pallas import tpu_sc as plsc

N_NODES = 10000
N_EDGES = 320000
D_IN = 128
D_HID = 32

NC, NS = 2, 16
NW = NC * NS
CHUNK = 125
EPW = N_EDGES // NW
NCHUNK = EPW // CHUNK
ACC_N = N_NODES
NPACKA = ACC_N * 32 // 128
NBUF = 8
AHEAD = 4

def _sc_mesh():
    return plsc.VectorSubcoreMesh(core_axis_name="c", subcore_axis_name="s",
                                  num_cores=NC, num_subcores=NS)


def _seg_sum_sc(table, src3, dst3, z32):
    scratch = [
        pltpu.VMEM((NCHUNK, CHUNK), jnp.int32),
        pltpu.VMEM((NCHUNK, CHUNK), jnp.int32),
        [pltpu.VMEM((CHUNK, 32), jnp.float32)] * NBUF,
        [pltpu.SemaphoreType.DMA] * NBUF,
        [pltpu.SemaphoreType.DMA] * NBUF,
        pltpu.VMEM_SHARED((ACC_N, 32), jnp.float32),
    ]

    def body(table_hbm, src_hbm, dst_hbm, z32_hbm, sums_hbm,
             sidx, didx, rows, gsem, ssem, acc):
        cid = lax.axis_index("c")
        sid = lax.axis_index("s")
        wid = sid * NC + cid

        @pl.when(sid == 0)
        def _():
            pltpu.sync_copy(z32_hbm, acc)

        pltpu.sync_copy(src_hbm.at[wid], sidx)
        pltpu.sync_copy(dst_hbm.at[wid], didx)
        plsc.subcore_barrier()

        for k in range(AHEAD):
            pltpu.async_copy(table_hbm.at[sidx.at[k]], rows[k], gsem[k])

        @pl.loop(0, NCHUNK, step=NBUF)
        def _(t):
            for k in range(NBUF):
                tt = t + k
                nb = (k + AHEAD) % NBUF
                pltpu.make_async_copy(
                    table_hbm.at[sidx.at[tt]], rows[k], gsem[k]).wait()
                pltpu.async_copy(rows[k], acc.at[didx.at[tt]], ssem[k],
                                 add=True)

                @pl.when(tt + AHEAD < NCHUNK)
                def _():
                    @pl.when(tt >= NBUF - AHEAD)
                    def _():
                        pltpu.make_async_copy(
                            rows[nb], acc.at[didx.at[tt - AHEAD]],
                            ssem[nb]).wait()
                    pltpu.async_copy(
                        table_hbm.at[sidx.at[tt + AHEAD]], rows[nb], gsem[nb])

        for k in range(NBUF):
            tt = NCHUNK - NBUF + k
            pltpu.make_async_copy(
                rows[k], acc.at[didx.at[tt]], ssem[k]).wait()

        plsc.subcore_barrier()

        @pl.when(sid == 0)
        def _():
            pltpu.sync_copy(acc, sums_hbm.at[cid])

    kern = pl.kernel(
        body, out_type=jax.ShapeDtypeStruct((NC, ACC_N, 32), jnp.float32),
        mesh=_sc_mesh(), scratch_types=scratch,
        compiler_params=pltpu.CompilerParams(use_tc_tiling_on_sc=False))
    return kern(table, src3, dst3, z32)


def _sc_counts(dst3, z32):
    scratch = [
        pltpu.VMEM((NCHUNK, CHUNK), jnp.int32),
        pltpu.VMEM((CHUNK, 32), jnp.float32),
        pltpu.VMEM_SHARED((ACC_N, 32), jnp.float32),
        pltpu.SemaphoreType.DMA,
    ]

    def body(dst_hbm, z32_hbm, cnts_hbm, didx, ones_v, cacc, csem):
        cid = lax.axis_index("c")
        sid = lax.axis_index("s")
        wid = sid * NC + cid

        @pl.when(sid == 0)
        def _():
            pltpu.sync_copy(z32_hbm, cacc)

        @pl.loop(0, CHUNK)
        def _(j):
            ones_v[j, 0:16] = jnp.full((16,), 1.0, jnp.float32)
            ones_v[j, 16:32] = jnp.full((16,), 1.0, jnp.float32)

        pltpu.sync_copy(dst_hbm.at[wid], didx)
        plsc.subcore_barrier()

        @pl.loop(0, NCHUNK)
        def _(t):
            pltpu.async_copy(ones_v, cacc.at[didx.at[t]], csem, add=True)

            @pl.when(t >= NBUF)
            def _():
                pltpu.make_async_copy(
                    ones_v, cacc.at[didx.at[t - NBUF]], csem).wait()

        for k in range(NBUF):
            tt = NCHUNK - NBUF + k
            pltpu.make_async_copy(ones_v, cacc.at[didx.at[tt]], csem).wait()

        plsc.subcore_barrier()

        @pl.when(sid == 0)
        def _():
            pltpu.sync_copy(cacc, cnts_hbm.at[cid])

    kern = pl.kernel(
        body, out_type=jax.ShapeDtypeStruct((NC, ACC_N, 32), jnp.float32),
        mesh=_sc_mesh(), scratch_types=scratch,
        compiler_params=pltpu.CompilerParams(use_tc_tiling_on_sc=False))
    return kern(dst3, z32)


def _tc_project(x, wcat):
    n, dout = x.shape[0], wcat.shape[1] // 2

    def body(x_ref, w_ref, p_ref, r_ref):
        xw = jnp.dot(x_ref[...], w_ref[...], preferred_element_type=jnp.float32)
        p_ref[...] = xw[:, :dout]
        r_ref[...] = xw[:, dout:]

    return pl.pallas_call(
        body,
        out_shape=(jax.ShapeDtypeStruct((n, dout), jnp.float32),
                   jax.ShapeDtypeStruct((n, dout), jnp.float32)),
    )(x, wcat)


NPACK = N_NODES // 4


def _tc_combine1(sums, cnts, r1p, b1p, w2blk):
    def body(s_ref, c_ref, r_ref, b_ref, w_ref, p_ref, q_ref, rc_ref):
        rinv = 1.0 / jnp.maximum(c_ref[0, :NPACK] + c_ref[1, :NPACK], 1.0)
        rc_ref[...] = rinv
        h = jnp.maximum(
            (s_ref[0, :NPACK] + s_ref[1, :NPACK]) * rinv + r_ref[...]
            + b_ref[...], 0.0)
        hw = jnp.dot(h, w_ref[...], preferred_element_type=jnp.float32)
        p_ref[...] = hw[:, :128]
        q_ref[...] = hw[:, 128:]

    return pl.pallas_call(
        body,
        out_shape=(jax.ShapeDtypeStruct((NPACK, 128), jnp.float32),
                   jax.ShapeDtypeStruct((NPACK, 128), jnp.float32),
                   jax.ShapeDtypeStruct((NPACK, 128), jnp.float32)),
    )(sums, cnts, r1p, b1p, w2blk)


def _tc_combine2(sums, rcp, r2p, b2p):
    def body(s_ref, rc_ref, r_ref, b_ref, out_ref):
        out_ref[...] = ((s_ref[0, :NPACK] + s_ref[1, :NPACK]) * rc_ref[...]
                        + r_ref[...] + b_ref[...])

    return pl.pallas_call(
        body,
        out_shape=jax.ShapeDtypeStruct((NPACK, 128), jnp.float32),
    )(sums, rcp, r2p, b2p)


def kernel(x, edge_index, W1_l, W1_r, b1, W2_l, W2_r, b2):
    src3 = edge_index[0].astype(jnp.int32).reshape(NW, NCHUNK, CHUNK)
    dst3 = edge_index[1].astype(jnp.int32).reshape(NW, NCHUNK, CHUNK)
    z32 = jnp.zeros((ACC_N, 32), jnp.float32)
    w1cat = jnp.concatenate([W1_l, W1_r], axis=1)
    eye4 = jnp.eye(4, dtype=jnp.float32)
    w2blk = jnp.concatenate([jnp.kron(eye4, W2_l),
                             jnp.kron(eye4, W2_r)], axis=1)
    b1p = jnp.tile(b1, 4).reshape(1, 128)
    b2p = jnp.tile(b2, 4).reshape(1, 128)

    cnts = _sc_counts(dst3, z32)
    p1, r1 = _tc_project(x, w1cat)
    sums1 = _seg_sum_sc(p1, src3, dst3, z32)
    p2p, r2p, rcp = _tc_combine1(sums1.reshape(NC, NPACKA, 128),
                                 cnts.reshape(NC, NPACKA, 128),
                                 r1.reshape(NPACK, 128), b1p, w2blk)
    sums2 = _seg_sum_sc(p2p.reshape(N_NODES, D_HID), src3, dst3, z32)
    outp = _tc_combine2(sums2.reshape(NC, NPACKA, 128), rcp, r2p, b2p)
    return outp.reshape(N_NODES, D_HID)


# speedup vs baseline: 1.0319x; 1.0319x over previous
import jax
import jax.numpy as jnp
from jax import lax
from jax.experimental import pallas as pl
from jax.experimental.pallas import tpu as pltpu
from jax.experimental.pallas import tpu_sc as plsc

N_NODES = 10000
N_EDGES = 320000
D_IN = 128
D_HID = 32

NC, NS = 2, 16
NW = NC * NS
CHUNK = 125
EPW = N_EDGES // NW
NCHUNK = EPW // CHUNK
ACC_N = N_NODES
NPACKA = ACC_N * 32 // 128
NBUF = 8
AHEAD = 4

def _sc_mesh():
    return plsc.VectorSubcoreMesh(core_axis_name="c", subcore_axis_name="s",
                                  num_cores=NC, num_subcores=NS)


def _seg_sum_sc(table, src3, dst3, z32):
    scratch = [
        pltpu.VMEM((NCHUNK, CHUNK), jnp.int32),
        pltpu.VMEM((NCHUNK, CHUNK), jnp.int32),
        [pltpu.VMEM((CHUNK, 32), jnp.float32)] * NBUF,
        [pltpu.SemaphoreType.DMA] * NBUF,
        [pltpu.SemaphoreType.DMA] * NBUF,
        pltpu.VMEM_SHARED((ACC_N, 32), jnp.float32),
    ]

    def body(table_hbm, src_hbm, dst_hbm, z32_hbm, sums_hbm,
             sidx, didx, rows, gsem, ssem, acc):
        cid = lax.axis_index("c")
        sid = lax.axis_index("s")
        wid = sid * NC + cid

        @pl.when(sid == 0)
        def _():
            pltpu.sync_copy(z32_hbm, acc)

        pltpu.sync_copy(src_hbm.at[wid], sidx)
        pltpu.sync_copy(dst_hbm.at[wid], didx)
        plsc.subcore_barrier()

        for k in range(AHEAD):
            pltpu.async_copy(table_hbm.at[sidx.at[k]], rows[k], gsem[k])

        @pl.loop(0, NCHUNK, step=NBUF)
        def _(t):
            for k in range(NBUF):
                tt = t + k
                nb = (k + AHEAD) % NBUF
                pltpu.make_async_copy(
                    table_hbm.at[sidx.at[tt]], rows[k], gsem[k]).wait()
                pltpu.async_copy(rows[k], acc.at[didx.at[tt]], ssem[k],
                                 add=True)

                @pl.when(tt + AHEAD < NCHUNK)
                def _():
                    @pl.when(tt >= NBUF - AHEAD)
                    def _():
                        pltpu.make_async_copy(
                            rows[nb], acc.at[didx.at[tt - AHEAD]],
                            ssem[nb]).wait()
                    pltpu.async_copy(
                        table_hbm.at[sidx.at[tt + AHEAD]], rows[nb], gsem[nb])

        for k in range(NBUF):
            tt = NCHUNK - NBUF + k
            pltpu.make_async_copy(
                rows[k], acc.at[didx.at[tt]], ssem[k]).wait()

        plsc.subcore_barrier()

        @pl.when(sid == 0)
        def _():
            pltpu.sync_copy(acc, sums_hbm.at[cid])

    kern = pl.kernel(
        body, out_type=jax.ShapeDtypeStruct((NC, ACC_N, 32), jnp.float32),
        mesh=_sc_mesh(), scratch_types=scratch,
        compiler_params=pltpu.CompilerParams(use_tc_tiling_on_sc=False))
    return kern(table, src3, dst3, z32)


def _sc_counts(dst3, z32):
    scratch = [
        pltpu.VMEM((NCHUNK, CHUNK), jnp.int32),
        pltpu.VMEM((CHUNK, 32), jnp.float32),
        pltpu.VMEM_SHARED((ACC_N, 32), jnp.float32),
        pltpu.SemaphoreType.DMA,
    ]

    def body(dst_hbm, z32_hbm, cnts_hbm, didx, ones_v, cacc, csem):
        cid = lax.axis_index("c")
        sid = lax.axis_index("s")
        wid = sid * NC + cid

        @pl.when(sid == 0)
        def _():
            pltpu.sync_copy(z32_hbm, cacc)

        @pl.loop(0, CHUNK)
        def _(j):
            ones_v[j, 0:16] = jnp.full((16,), 1.0, jnp.float32)
            ones_v[j, 16:32] = jnp.full((16,), 1.0, jnp.float32)

        pltpu.sync_copy(dst_hbm.at[wid], didx)
        plsc.subcore_barrier()

        @pl.loop(0, NCHUNK)
        def _(t):
            pltpu.async_copy(ones_v, cacc.at[didx.at[t]], csem, add=True)

            @pl.when(t >= NBUF)
            def _():
                pltpu.make_async_copy(
                    ones_v, cacc.at[didx.at[t - NBUF]], csem).wait()

        for k in range(NBUF):
            tt = NCHUNK - NBUF + k
            pltpu.make_async_copy(ones_v, cacc.at[didx.at[tt]], csem).wait()

        plsc.subcore_barrier()

        @pl.when(sid == 0)
        def _():
            pltpu.sync_copy(cacc, cnts_hbm.at[cid])

    kern = pl.kernel(
        body, out_type=jax.ShapeDtypeStruct((NC, ACC_N, 32), jnp.float32),
        mesh=_sc_mesh(), scratch_types=scratch,
        compiler_params=pltpu.CompilerParams(use_tc_tiling_on_sc=False))
    return kern(dst3, z32)


def _tc_project(x, wcat):
    n, dout = x.shape[0], wcat.shape[1] // 2

    def body(x_ref, w_ref, p_ref, r_ref):
        xw = jnp.dot(x_ref[...], w_ref[...], preferred_element_type=jnp.float32)
        p_ref[...] = xw[:, :dout]
        r_ref[...] = xw[:, dout:]

    return pl.pallas_call(
        body,
        out_shape=(jax.ShapeDtypeStruct((n, dout), jnp.float32),
                   jax.ShapeDtypeStruct((n, dout), jnp.float32)),
    )(x, wcat)


NPACK = N_NODES // 4


def _tc_combine1(sums, cnts, r1p, b1p, w2blk):
    def body(s_ref, c_ref, r_ref, b_ref, w_ref, p_ref, q_ref, rc_ref):
        rinv = 1.0 / jnp.maximum(c_ref[0, :NPACK] + c_ref[1, :NPACK], 1.0)
        rc_ref[...] = rinv
        h = jnp.maximum(
            (s_ref[0, :NPACK] + s_ref[1, :NPACK]) * rinv + r_ref[...]
            + b_ref[...], 0.0)
        hw = jnp.dot(h, w_ref[...], preferred_element_type=jnp.float32)
        p_ref[...] = hw[:, :128]
        q_ref[...] = hw[:, 128:]

    return pl.pallas_call(
        body,
        out_shape=(jax.ShapeDtypeStruct((NPACK, 128), jnp.float32),
                   jax.ShapeDtypeStruct((NPACK, 128), jnp.float32),
                   jax.ShapeDtypeStruct((NPACK, 128), jnp.float32)),
    )(sums, cnts, r1p, b1p, w2blk)


def _tc_combine2(sums, rcp, r2p, b2p):
    def body(s_ref, rc_ref, r_ref, b_ref, out_ref):
        out_ref[...] = ((s_ref[0, :NPACK] + s_ref[1, :NPACK]) * rc_ref[...]
                        + r_ref[...] + b_ref[...])

    return pl.pallas_call(
        body,
        out_shape=jax.ShapeDtypeStruct((NPACK, 128), jnp.float32),
    )(sums, rcp, r2p, b2p)


def kernel(x, edge_index, W1_l, W1_r, b1, W2_l, W2_r, b2):
    src3 = edge_index[0].astype(jnp.int32).reshape(NW, NCHUNK, CHUNK)
    dst3 = edge_index[1].astype(jnp.int32).reshape(NW, NCHUNK, CHUNK)
    z32 = jnp.zeros((ACC_N, 32), jnp.float32)
    w1cat = jnp.concatenate([W1_l, W1_r], axis=1)
    eye4 = jnp.eye(4, dtype=jnp.float32)
    w2blk = jnp.concatenate([jnp.kron(eye4, W2_l),
                             jnp.kron(eye4, W2_r)], axis=1)
    b1p = jnp.tile(b1, 4).reshape(1, 128)
    b2p = jnp.tile(b2, 4).reshape(1, 128)

    cnts = _sc_counts(dst3, z32)
    p1, r1 = _tc_project(x, w1cat)
    p1, cnts = lax.optimization_barrier((p1, cnts))
    sums1 = _seg_sum_sc(p1, src3, dst3, z32)
    p2p, r2p, rcp = _tc_combine1(sums1.reshape(NC, NPACKA, 128),
                                 cnts.reshape(NC, NPACKA, 128),
                                 r1.reshape(NPACK, 128), b1p, w2blk)
    sums2 = _seg_sum_sc(p2p.reshape(N_NODES, D_HID), src3, dst3, z32)
    outp = _tc_combine2(sums2.reshape(NC, NPACKA, 128), rcp, r2p, b2p)
    return outp.reshape(N_NODES, D_HID)


# speedup vs baseline: 1.0485x; 1.0160x over previous
import jax
import jax.numpy as jnp
from jax import lax
from jax.experimental import pallas as pl
from jax.experimental.pallas import tpu as pltpu
from jax.experimental.pallas import tpu_sc as plsc

N_NODES = 10000
N_EDGES = 320000
D_IN = 128
D_HID = 32

NC, NS = 2, 16
NW = NC * NS
CHUNK = 125
EPW = N_EDGES // NW
NCHUNK = EPW // CHUNK
ACC_N = N_NODES
NPACKA = ACC_N * 32 // 128
NBUF = 8
AHEAD = 4

def _sc_mesh():
    return plsc.VectorSubcoreMesh(core_axis_name="c", subcore_axis_name="s",
                                  num_cores=NC, num_subcores=NS)


def _seg_sum_sc(table, src3, dst3, z32):
    scratch = [
        pltpu.VMEM((NCHUNK, CHUNK), jnp.int32),
        pltpu.VMEM((NCHUNK, CHUNK), jnp.int32),
        [pltpu.VMEM((CHUNK, 32), jnp.float32)] * NBUF,
        [pltpu.SemaphoreType.DMA] * NBUF,
        [pltpu.SemaphoreType.DMA] * NBUF,
        pltpu.VMEM_SHARED((ACC_N, 32), jnp.float32),
    ]

    def body(table_hbm, src_hbm, dst_hbm, z32_hbm, sums_hbm,
             sidx, didx, rows, gsem, ssem, acc):
        cid = lax.axis_index("c")
        sid = lax.axis_index("s")
        wid = sid * NC + cid

        @pl.when(sid == 0)
        def _():
            pltpu.sync_copy(z32_hbm, acc)

        pltpu.sync_copy(src_hbm.at[wid], sidx)
        pltpu.sync_copy(dst_hbm.at[wid], didx)
        plsc.subcore_barrier()

        for k in range(AHEAD):
            pltpu.async_copy(table_hbm.at[sidx.at[k]], rows[k], gsem[k])

        @pl.loop(0, NCHUNK, step=NBUF)
        def _(t):
            for k in range(NBUF):
                tt = t + k
                nb = (k + AHEAD) % NBUF
                pltpu.make_async_copy(
                    table_hbm.at[sidx.at[tt]], rows[k], gsem[k]).wait()
                pltpu.async_copy(rows[k], acc.at[didx.at[tt]], ssem[k],
                                 add=True)

                @pl.when(tt + AHEAD < NCHUNK)
                def _():
                    @pl.when(tt >= NBUF - AHEAD)
                    def _():
                        pltpu.make_async_copy(
                            rows[nb], acc.at[didx.at[tt - AHEAD]],
                            ssem[nb]).wait()
                    pltpu.async_copy(
                        table_hbm.at[sidx.at[tt + AHEAD]], rows[nb], gsem[nb])

        for k in range(NBUF):
            tt = NCHUNK - NBUF + k
            pltpu.make_async_copy(
                rows[k], acc.at[didx.at[tt]], ssem[k]).wait()

        plsc.subcore_barrier()

        @pl.when(sid == 0)
        def _():
            pltpu.sync_copy(acc, sums_hbm.at[cid])

    kern = pl.kernel(
        body, out_type=jax.ShapeDtypeStruct((NC, ACC_N, 32), jnp.float32),
        mesh=_sc_mesh(), scratch_types=scratch,
        compiler_params=pltpu.CompilerParams(use_tc_tiling_on_sc=False))
    return kern(table, src3, dst3, z32)


def _sc_counts(dst3, z32):
    scratch = [
        pltpu.VMEM((NCHUNK, CHUNK), jnp.int32),
        pltpu.VMEM((CHUNK, 32), jnp.float32),
        pltpu.VMEM_SHARED((ACC_N, 32), jnp.float32),
        pltpu.SemaphoreType.DMA,
    ]

    def body(dst_hbm, z32_hbm, cnts_hbm, didx, ones_v, cacc, csem):
        cid = lax.axis_index("c")
        sid = lax.axis_index("s")
        wid = sid * NC + cid

        @pl.when(sid == 0)
        def _():
            pltpu.sync_copy(z32_hbm, cacc)

        @pl.loop(0, CHUNK)
        def _(j):
            ones_v[j, 0:16] = jnp.full((16,), 1.0, jnp.float32)
            ones_v[j, 16:32] = jnp.full((16,), 1.0, jnp.float32)

        pltpu.sync_copy(dst_hbm.at[wid], didx)
        plsc.subcore_barrier()

        @pl.loop(0, NCHUNK)
        def _(t):
            pltpu.async_copy(ones_v, cacc.at[didx.at[t]], csem, add=True)

            @pl.when(t >= NBUF)
            def _():
                pltpu.make_async_copy(
                    ones_v, cacc.at[didx.at[t - NBUF]], csem).wait()

        for k in range(NBUF):
            tt = NCHUNK - NBUF + k
            pltpu.make_async_copy(ones_v, cacc.at[didx.at[tt]], csem).wait()

        plsc.subcore_barrier()

        @pl.when(sid == 0)
        def _():
            pltpu.sync_copy(cacc, cnts_hbm.at[cid])

    kern = pl.kernel(
        body, out_type=jax.ShapeDtypeStruct((NC, ACC_N, 32), jnp.float32),
        mesh=_sc_mesh(), scratch_types=scratch,
        compiler_params=pltpu.CompilerParams(use_tc_tiling_on_sc=False))
    return kern(dst3, z32)


def _tc_project(x, wcat):
    n, dout = x.shape[0], wcat.shape[1] // 2

    def body(x_ref, w_ref, p_ref, r_ref):
        xw = jnp.dot(x_ref[...], w_ref[...], preferred_element_type=jnp.float32)
        p_ref[...] = xw[:, :dout]
        r_ref[...] = xw[:, dout:]

    return pl.pallas_call(
        body,
        out_shape=(jax.ShapeDtypeStruct((n, dout), jnp.float32),
                   jax.ShapeDtypeStruct((n, dout), jnp.float32)),
    )(x, wcat)


NPACK = N_NODES // 4


def _tc_combine1(sums, cnts, r1p, b1p, w2blk):
    def body(s_ref, c_ref, r_ref, b_ref, w_ref, p_ref, q_ref, rc_ref):
        rinv = 1.0 / jnp.maximum(c_ref[0, :NPACK] + c_ref[1, :NPACK], 1.0)
        rc_ref[...] = rinv
        h = jnp.maximum(
            (s_ref[0, :NPACK] + s_ref[1, :NPACK]) * rinv + r_ref[...]
            + b_ref[...], 0.0)
        hw = jnp.dot(h, w_ref[...], preferred_element_type=jnp.float32)
        p_ref[...] = hw[:, :128]
        q_ref[...] = hw[:, 128:]

    return pl.pallas_call(
        body,
        out_shape=(jax.ShapeDtypeStruct((NPACK, 128), jnp.float32),
                   jax.ShapeDtypeStruct((NPACK, 128), jnp.float32),
                   jax.ShapeDtypeStruct((NPACK, 128), jnp.float32)),
    )(sums, cnts, r1p, b1p, w2blk)


def _tc_combine2(sums, rcp, r2p, b2p):
    def body(s_ref, rc_ref, r_ref, b_ref, out_ref):
        out_ref[...] = ((s_ref[0, :NPACK] + s_ref[1, :NPACK]) * rc_ref[...]
                        + r_ref[...] + b_ref[...])

    return pl.pallas_call(
        body,
        out_shape=jax.ShapeDtypeStruct((NPACK, 128), jnp.float32),
    )(sums, rcp, r2p, b2p)


def kernel(x, edge_index, W1_l, W1_r, b1, W2_l, W2_r, b2):
    src3 = edge_index[0].astype(jnp.int32).reshape(NW, NCHUNK, CHUNK)
    dst3 = edge_index[1].astype(jnp.int32).reshape(NW, NCHUNK, CHUNK)
    z32 = jnp.zeros((ACC_N, 32), jnp.float32)
    w1cat = jnp.concatenate([W1_l, W1_r], axis=1)
    eye4 = jnp.eye(4, dtype=jnp.float32)
    w2blk = jnp.concatenate([jnp.kron(eye4, W2_l),
                             jnp.kron(eye4, W2_r)], axis=1)
    b1p = jnp.tile(b1, 4).reshape(1, 128)
    b2p = jnp.tile(b2, 4).reshape(1, 128)

    cnts = _sc_counts(dst3, z32)
    p1, r1 = _tc_project(x, w1cat)
    src3b, cnts = lax.optimization_barrier((src3, cnts))
    sums1 = _seg_sum_sc(p1, src3b, dst3, z32)
    p2p, r2p, rcp = _tc_combine1(sums1.reshape(NC, NPACKA, 128),
                                 cnts.reshape(NC, NPACKA, 128),
                                 r1.reshape(NPACK, 128), b1p, w2blk)
    sums2 = _seg_sum_sc(p2p.reshape(N_NODES, D_HID), src3, dst3, z32)
    outp = _tc_combine2(sums2.reshape(NC, NPACKA, 128), rcp, r2p, b2p)
    return outp.reshape(N_NODES, D_HID)


# speedup vs baseline: 1.0655x; 1.0163x over previous
import jax
import jax.numpy as jnp
from jax import lax
from jax.experimental import pallas as pl
from jax.experimental.pallas import tpu as pltpu
from jax.experimental.pallas import tpu_sc as plsc

N_NODES = 10000
N_EDGES = 320000
D_IN = 128
D_HID = 32

NC, NS = 2, 16
NW = NC * NS
CHUNK = 125
EPW = N_EDGES // NW
NCHUNK = EPW // CHUNK
ACC_N = N_NODES
NPACKA = ACC_N * 32 // 128
NBUF = 10
AHEAD = 5

def _sc_mesh():
    return plsc.VectorSubcoreMesh(core_axis_name="c", subcore_axis_name="s",
                                  num_cores=NC, num_subcores=NS)


def _seg_sum_sc(table, src3, dst3, z32):
    scratch = [
        pltpu.VMEM((NCHUNK, CHUNK), jnp.int32),
        pltpu.VMEM((NCHUNK, CHUNK), jnp.int32),
        [pltpu.VMEM((CHUNK, 32), jnp.float32)] * NBUF,
        [pltpu.SemaphoreType.DMA] * NBUF,
        [pltpu.SemaphoreType.DMA] * NBUF,
        pltpu.VMEM_SHARED((ACC_N, 32), jnp.float32),
    ]

    def body(table_hbm, src_hbm, dst_hbm, z32_hbm, sums_hbm,
             sidx, didx, rows, gsem, ssem, acc):
        cid = lax.axis_index("c")
        sid = lax.axis_index("s")
        wid = sid * NC + cid

        @pl.when(sid == 0)
        def _():
            pltpu.sync_copy(z32_hbm, acc)

        pltpu.sync_copy(src_hbm.at[wid], sidx)
        pltpu.sync_copy(dst_hbm.at[wid], didx)
        plsc.subcore_barrier()

        for k in range(AHEAD):
            pltpu.async_copy(table_hbm.at[sidx.at[k]], rows[k], gsem[k])

        @pl.loop(0, NCHUNK, step=NBUF)
        def _(t):
            for k in range(NBUF):
                tt = t + k
                nb = (k + AHEAD) % NBUF
                pltpu.make_async_copy(
                    table_hbm.at[sidx.at[tt]], rows[k], gsem[k]).wait()
                pltpu.async_copy(rows[k], acc.at[didx.at[tt]], ssem[k],
                                 add=True)

                @pl.when(tt + AHEAD < NCHUNK)
                def _():
                    @pl.when(tt >= NBUF - AHEAD)
                    def _():
                        pltpu.make_async_copy(
                            rows[nb], acc.at[didx.at[tt - AHEAD]],
                            ssem[nb]).wait()
                    pltpu.async_copy(
                        table_hbm.at[sidx.at[tt + AHEAD]], rows[nb], gsem[nb])

        for k in range(NBUF):
            tt = NCHUNK - NBUF + k
            pltpu.make_async_copy(
                rows[k], acc.at[didx.at[tt]], ssem[k]).wait()

        plsc.subcore_barrier()

        @pl.when(sid == 0)
        def _():
            pltpu.sync_copy(acc, sums_hbm.at[cid])

    kern = pl.kernel(
        body, out_type=jax.ShapeDtypeStruct((NC, ACC_N, 32), jnp.float32),
        mesh=_sc_mesh(), scratch_types=scratch,
        compiler_params=pltpu.CompilerParams(use_tc_tiling_on_sc=False))
    return kern(table, src3, dst3, z32)


def _sc_counts(dst3, z32):
    scratch = [
        pltpu.VMEM((NCHUNK, CHUNK), jnp.int32),
        pltpu.VMEM((CHUNK, 32), jnp.float32),
        pltpu.VMEM_SHARED((ACC_N, 32), jnp.float32),
        pltpu.SemaphoreType.DMA,
    ]

    def body(dst_hbm, z32_hbm, cnts_hbm, didx, ones_v, cacc, csem):
        cid = lax.axis_index("c")
        sid = lax.axis_index("s")
        wid = sid * NC + cid

        @pl.when(sid == 0)
        def _():
            pltpu.sync_copy(z32_hbm, cacc)

        @pl.loop(0, CHUNK)
        def _(j):
            ones_v[j, 0:16] = jnp.full((16,), 1.0, jnp.float32)
            ones_v[j, 16:32] = jnp.full((16,), 1.0, jnp.float32)

        pltpu.sync_copy(dst_hbm.at[wid], didx)
        plsc.subcore_barrier()

        @pl.loop(0, NCHUNK)
        def _(t):
            pltpu.async_copy(ones_v, cacc.at[didx.at[t]], csem, add=True)

            @pl.when(t >= NBUF)
            def _():
                pltpu.make_async_copy(
                    ones_v, cacc.at[didx.at[t - NBUF]], csem).wait()

        for k in range(NBUF):
            tt = NCHUNK - NBUF + k
            pltpu.make_async_copy(ones_v, cacc.at[didx.at[tt]], csem).wait()

        plsc.subcore_barrier()

        @pl.when(sid == 0)
        def _():
            pltpu.sync_copy(cacc, cnts_hbm.at[cid])

    kern = pl.kernel(
        body, out_type=jax.ShapeDtypeStruct((NC, ACC_N, 32), jnp.float32),
        mesh=_sc_mesh(), scratch_types=scratch,
        compiler_params=pltpu.CompilerParams(use_tc_tiling_on_sc=False))
    return kern(dst3, z32)


def _tc_project(x, wcat):
    n, dout = x.shape[0], wcat.shape[1] // 2

    def body(x_ref, w_ref, p_ref, r_ref):
        xw = jnp.dot(x_ref[...], w_ref[...], preferred_element_type=jnp.float32)
        p_ref[...] = xw[:, :dout]
        r_ref[...] = xw[:, dout:]

    return pl.pallas_call(
        body,
        out_shape=(jax.ShapeDtypeStruct((n, dout), jnp.float32),
                   jax.ShapeDtypeStruct((n, dout), jnp.float32)),
    )(x, wcat)


NPACK = N_NODES // 4


def _tc_combine1(sums, cnts, r1p, b1p, w2blk):
    def body(s_ref, c_ref, r_ref, b_ref, w_ref, p_ref, q_ref, rc_ref):
        rinv = 1.0 / jnp.maximum(c_ref[0, :NPACK] + c_ref[1, :NPACK], 1.0)
        rc_ref[...] = rinv
        h = jnp.maximum(
            (s_ref[0, :NPACK] + s_ref[1, :NPACK]) * rinv + r_ref[...]
            + b_ref[...], 0.0)
        hw = jnp.dot(h, w_ref[...], preferred_element_type=jnp.float32)
        p_ref[...] = hw[:, :128]
        q_ref[...] = hw[:, 128:]

    return pl.pallas_call(
        body,
        out_shape=(jax.ShapeDtypeStruct((NPACK, 128), jnp.float32),
                   jax.ShapeDtypeStruct((NPACK, 128), jnp.float32),
                   jax.ShapeDtypeStruct((NPACK, 128), jnp.float32)),
    )(sums, cnts, r1p, b1p, w2blk)


def _tc_combine2(sums, rcp, r2p, b2p):
    def body(s_ref, rc_ref, r_ref, b_ref, out_ref):
        out_ref[...] = ((s_ref[0, :NPACK] + s_ref[1, :NPACK]) * rc_ref[...]
                        + r_ref[...] + b_ref[...])

    return pl.pallas_call(
        body,
        out_shape=jax.ShapeDtypeStruct((NPACK, 128), jnp.float32),
    )(sums, rcp, r2p, b2p)


def kernel(x, edge_index, W1_l, W1_r, b1, W2_l, W2_r, b2):
    src3 = edge_index[0].astype(jnp.int32).reshape(NW, NCHUNK, CHUNK)
    dst3 = edge_index[1].astype(jnp.int32).reshape(NW, NCHUNK, CHUNK)
    z32 = jnp.zeros((ACC_N, 32), jnp.float32)
    w1cat = jnp.concatenate([W1_l, W1_r], axis=1)
    eye4 = jnp.eye(4, dtype=jnp.float32)
    w2blk = jnp.concatenate([jnp.kron(eye4, W2_l),
                             jnp.kron(eye4, W2_r)], axis=1)
    b1p = jnp.tile(b1, 4).reshape(1, 128)
    b2p = jnp.tile(b2, 4).reshape(1, 128)

    cnts = _sc_counts(dst3, z32)
    p1, r1 = _tc_project(x, w1cat)
    src3b, cnts = lax.optimization_barrier((src3, cnts))
    sums1 = _seg_sum_sc(p1, src3b, dst3, z32)
    p2p, r2p, rcp = _tc_combine1(sums1.reshape(NC, NPACKA, 128),
                                 cnts.reshape(NC, NPACKA, 128),
                                 r1.reshape(NPACK, 128), b1p, w2blk)
    sums2 = _seg_sum_sc(p2p.reshape(N_NODES, D_HID), src3, dst3, z32)
    outp = _tc_combine2(sums2.reshape(NC, NPACKA, 128), rcp, r2p, b2p)
    return outp.reshape(N_NODES, D_HID)
